# baseline (speedup 1.0000x reference)
import jax
import jax.numpy as jnp
from jax.experimental import pallas as pl

BS, NUM_ROWS, NUM_XS, ENC_DIM = 4, 128, 64, 15
HEADS, OUT_PER_HEAD, NUM_LAYERS = 4, 4, 2
IN_DIM = 1 + ENC_DIM
HID = HEADS * OUT_PER_HEAD
NUM_CLASSES = 2
LANES = HEADS * NUM_XS
CLAMP = 60.0

ROW_TILE = 64


def _gat_kernel(xs_ref, xsl_ref, w0_ref, pe_ref, vc_ref, mcat2_ref, lin2_ref,
                bias_ref, linf_ref, out_ref):
    f32 = jnp.float32
    bf16 = jnp.bfloat16
    i32 = jnp.int32
    lane_src = jax.lax.broadcasted_iota(i32, (NUM_XS, LANES), 1) % NUM_XS
    node_idx = jax.lax.broadcasted_iota(i32, (NUM_XS, LANES), 0)
    dmask = (lane_src == node_idx).astype(f32)
    bsum = (jax.lax.broadcasted_iota(i32, (LANES, HEADS), 0) // NUM_XS ==
            jax.lax.broadcasted_iota(i32, (LANES, HEADS), 1)).astype(bf16)
    e16 = (jax.lax.broadcasted_iota(i32, (HEADS, HID), 0) ==
           jax.lax.broadcasted_iota(i32, (HEADS, HID), 1) // OUT_PER_HEAD
           ).astype(f32)
    ch = jax.lax.broadcasted_iota(i32, (1, 1, HID), 2) // OUT_PER_HEAD

    def attention(logits, xp3):
        lb = logits.astype(bf16)
        e_in = jnp.minimum(jnp.maximum(lb, bf16(0.2) * lb), bf16(CLAMP))
        e = jnp.exp(e_in)
        den = jax.lax.dot_general(
            e, bsum, (((2,), (0,)), ((), ())),
            preferred_element_type=f32)
        rcp = (1.0 / (den + 1e-16)) @ e16
        xp3h = xp3.astype(bf16)
        xstack = jnp.concatenate(
            [xp3h * (ch == h).astype(bf16) for h in range(HEADS)],
            axis=1)
        agg = jax.lax.dot_general(
            e, xstack, (((2,), (1,)), ((0,), (0,))),
            preferred_element_type=f32)
        return agg * rcp

    xsb = xs_ref[0]
    xsl = xsl_ref[0][:, None, :]
    w0 = w0_ref[0][None]
    pe = pe_ref[0][None]
    vc = vc_ref[0][None]

    a_dst_big1 = xsb * w0[:, :, :LANES] + pe[:, :, :LANES]
    xp31 = xsb * w0[:, :, LANES:] + pe[:, :, LANES:]
    v1 = xsl * vc
    x3 = attention(a_dst_big1 + v1, xp31) + bias_ref[0, 0]

    xp32 = jax.lax.dot_general(
        x3, lin2_ref[0], (((2,), (1,)), ((), ())),
        preferred_element_type=f32)
    xp32h = xp32.astype(bf16)
    tmp_src2 = jax.lax.dot_general(
        xp32h, mcat2_ref[0, :, :LANES], (((2,), (0,)), ((), ())),
        preferred_element_type=f32)
    a_dst_big2 = jax.lax.dot_general(
        xp32h, mcat2_ref[0, :, LANES:], (((2,), (0,)), ((), ())),
        preferred_element_type=f32)
    v2 = jnp.sum(tmp_src2 * dmask, axis=1, keepdims=True)
    x3 = attention(a_dst_big2 + v2, xp32) + bias_ref[0, 1]

    xsum = jnp.sum(x3, axis=1)
    out_ref[0] = jax.lax.dot_general(
        xsum, linf_ref[0], (((1,), (0,)), ((), ())),
        preferred_element_type=f32)


def kernel(xs, pos_enc, gat_lin, gat_src, gat_dst, gat_bias, lin_final):
    bs, num_rows, num_xs = xs.shape

    eye = jnp.eye(HEADS, dtype=xs.dtype)
    msrc = (gat_src[:, :, 0, :, :, None] * eye[:, None, :]).reshape(
        bs, NUM_LAYERS, HID, HEADS)
    mdst = (gat_dst[:, :, 0, :, :, None] * eye[:, None, :]).reshape(
        bs, NUM_LAYERS, HID, HEADS)
    msrc_e = jnp.repeat(msrc, NUM_XS, axis=-1)
    mdst_e = jnp.repeat(mdst, NUM_XS, axis=-1)
    lin_t = jnp.swapaxes(gat_lin, 2, 3)

    d2 = lin_t[:, 0] @ msrc[:, 0]
    d1 = lin_t[:, 0] @ mdst_e[:, 0]
    w1_0 = jnp.concatenate([d1[:, 0], lin_t[:, 0, 0]], axis=-1)
    pall = pos_enc @ jnp.concatenate(
        [d1[:, 1:], lin_t[:, 0, 1:], d2[:, 1:]], axis=-1)
    pe_v = jnp.swapaxes(pall[:, :, LANES + HID:], 1, 2).reshape(bs, LANES)
    pe_proj = jnp.concatenate(
        [pall[:, :, :LANES] + pe_v[:, None, :],
         pall[:, :, LANES:LANES + HID]], axis=-1)
    vc = jnp.repeat(d2[:, 0], num_xs, axis=-1)[:, None]
    xs_lanes = jnp.tile(xs, (1, 1, HEADS))

    mcat2 = jnp.concatenate([msrc_e[:, 1], mdst_e[:, 1]],
                            axis=-1).astype(jnp.bfloat16)
    bias = gat_bias.reshape(bs, NUM_LAYERS, 1, HID)
    linf = jnp.swapaxes(lin_final, 1, 2)

    r = ROW_TILE
    grid = (bs, num_rows // r)
    out = pl.pallas_call(
        _gat_kernel,
        grid=grid,
        in_specs=[
            pl.BlockSpec((1, r, num_xs, 1), lambda b, i: (b, i, 0, 0)),
            pl.BlockSpec((1, r, LANES), lambda b, i: (b, i, 0)),
            pl.BlockSpec((1, 1, LANES + HID), lambda b, i: (b, 0, 0)),
            pl.BlockSpec((1, num_xs, LANES + HID), lambda b, i: (b, 0, 0)),
            pl.BlockSpec((1, 1, LANES), lambda b, i: (b, 0, 0)),
            pl.BlockSpec((1, HID, 2 * LANES), lambda b, i: (b, 0, 0)),
            pl.BlockSpec((1, HID, HID), lambda b, i: (b, 0, 0)),
            pl.BlockSpec((1, NUM_LAYERS, 1, HID), lambda b, i: (b, 0, 0, 0)),
            pl.BlockSpec((1, HID, NUM_CLASSES), lambda b, i: (b, 0, 0)),
        ],
        out_specs=pl.BlockSpec((1, r, NUM_CLASSES), lambda b, i: (b, i, 0)),
        out_shape=jax.ShapeDtypeStruct((bs, num_rows, NUM_CLASSES), xs.dtype),
    )(xs[..., None], xs_lanes, w1_0[:, None], pe_proj, vc, mcat2,
      gat_lin[:, 1], bias, linf)
    return out


# speedup vs baseline: 1.0283x; 1.0283x over previous
import jax
import jax.numpy as jnp
from jax.experimental import pallas as pl

BS, NUM_ROWS, NUM_XS, ENC_DIM = 4, 128, 64, 15
HEADS, OUT_PER_HEAD, NUM_LAYERS = 4, 4, 2
IN_DIM = 1 + ENC_DIM
HID = HEADS * OUT_PER_HEAD
NUM_CLASSES = 2
LANES = HEADS * NUM_XS
CLAMP = 60.0

ROW_TILE = 128


def _gat_kernel(xs_ref, xsl_ref, w0_ref, pe_ref, vc_ref, mcat2_ref, lin2_ref,
                bias_ref, linf_ref, out_ref):
    f32 = jnp.float32
    bf16 = jnp.bfloat16
    i32 = jnp.int32
    lane_src = jax.lax.broadcasted_iota(i32, (NUM_XS, LANES), 1) % NUM_XS
    node_idx = jax.lax.broadcasted_iota(i32, (NUM_XS, LANES), 0)
    dmask = (lane_src == node_idx).astype(f32)
    bsum = (jax.lax.broadcasted_iota(i32, (LANES, HEADS), 0) // NUM_XS ==
            jax.lax.broadcasted_iota(i32, (LANES, HEADS), 1)).astype(bf16)
    e16 = (jax.lax.broadcasted_iota(i32, (HEADS, HID), 0) ==
           jax.lax.broadcasted_iota(i32, (HEADS, HID), 1) // OUT_PER_HEAD
           ).astype(f32)
    ch = jax.lax.broadcasted_iota(i32, (1, 1, HID), 2) // OUT_PER_HEAD

    def attention(logits, xp3):
        lb = logits.astype(bf16)
        e_in = jnp.minimum(jnp.maximum(lb, bf16(0.2) * lb), bf16(CLAMP))
        e = jnp.exp(e_in)
        den = jax.lax.dot_general(
            e, bsum, (((2,), (0,)), ((), ())),
            preferred_element_type=f32)
        rcp = (1.0 / (den + 1e-16)) @ e16
        xp3h = xp3.astype(bf16)
        xstack = jnp.concatenate(
            [xp3h * (ch == h).astype(bf16) for h in range(HEADS)],
            axis=1)
        agg = jax.lax.dot_general(
            e, xstack, (((2,), (1,)), ((0,), (0,))),
            preferred_element_type=f32)
        return agg * rcp

    xsb = xs_ref[0]
    xsl = xsl_ref[0][:, None, :]
    w0 = w0_ref[0][None]
    pe = pe_ref[0][None]
    vc = vc_ref[0][None]

    a_dst_big1 = xsb * w0[:, :, :LANES] + pe[:, :, :LANES]
    xp31 = xsb * w0[:, :, LANES:] + pe[:, :, LANES:]
    v1 = xsl * vc
    x3 = attention(a_dst_big1 + v1, xp31) + bias_ref[0, 0]

    xp32 = jax.lax.dot_general(
        x3, lin2_ref[0], (((2,), (1,)), ((), ())),
        preferred_element_type=f32)
    xp32h = xp32.astype(bf16)
    tmp_src2 = jax.lax.dot_general(
        xp32h, mcat2_ref[0, :, :LANES], (((2,), (0,)), ((), ())),
        preferred_element_type=f32)
    a_dst_big2 = jax.lax.dot_general(
        xp32h, mcat2_ref[0, :, LANES:], (((2,), (0,)), ((), ())),
        preferred_element_type=f32)
    v2 = jnp.sum(tmp_src2 * dmask, axis=1, keepdims=True)
    x3 = attention(a_dst_big2 + v2, xp32) + bias_ref[0, 1]

    xsum = jnp.sum(x3, axis=1)
    out_ref[0] = jax.lax.dot_general(
        xsum, linf_ref[0], (((1,), (0,)), ((), ())),
        preferred_element_type=f32)


def kernel(xs, pos_enc, gat_lin, gat_src, gat_dst, gat_bias, lin_final):
    bs, num_rows, num_xs = xs.shape

    eye = jnp.eye(HEADS, dtype=xs.dtype)
    msrc = (gat_src[:, :, 0, :, :, None] * eye[:, None, :]).reshape(
        bs, NUM_LAYERS, HID, HEADS)
    mdst = (gat_dst[:, :, 0, :, :, None] * eye[:, None, :]).reshape(
        bs, NUM_LAYERS, HID, HEADS)
    msrc_e = jnp.repeat(msrc, NUM_XS, axis=-1)
    mdst_e = jnp.repeat(mdst, NUM_XS, axis=-1)
    lin_t = jnp.swapaxes(gat_lin, 2, 3)

    d2 = lin_t[:, 0] @ msrc[:, 0]
    d1 = lin_t[:, 0] @ mdst_e[:, 0]
    w1_0 = jnp.concatenate([d1[:, 0], lin_t[:, 0, 0]], axis=-1)
    pall = pos_enc @ jnp.concatenate(
        [d1[:, 1:], lin_t[:, 0, 1:], d2[:, 1:]], axis=-1)
    pe_v = jnp.swapaxes(pall[:, :, LANES + HID:], 1, 2).reshape(bs, LANES)
    pe_proj = jnp.concatenate(
        [pall[:, :, :LANES] + pe_v[:, None, :],
         pall[:, :, LANES:LANES + HID]], axis=-1)
    vc = jnp.repeat(d2[:, 0], num_xs, axis=-1)[:, None]
    xs_lanes = jnp.tile(xs, (1, 1, HEADS))

    mcat2 = jnp.concatenate([msrc_e[:, 1], mdst_e[:, 1]],
                            axis=-1).astype(jnp.bfloat16)
    bias = gat_bias.reshape(bs, NUM_LAYERS, 1, HID)
    linf = jnp.swapaxes(lin_final, 1, 2)

    r = ROW_TILE
    grid = (bs, num_rows // r)
    out = pl.pallas_call(
        _gat_kernel,
        grid=grid,
        in_specs=[
            pl.BlockSpec((1, r, num_xs, 1), lambda b, i: (b, i, 0, 0)),
            pl.BlockSpec((1, r, LANES), lambda b, i: (b, i, 0)),
            pl.BlockSpec((1, 1, LANES + HID), lambda b, i: (b, 0, 0)),
            pl.BlockSpec((1, num_xs, LANES + HID), lambda b, i: (b, 0, 0)),
            pl.BlockSpec((1, 1, LANES), lambda b, i: (b, 0, 0)),
            pl.BlockSpec((1, HID, 2 * LANES), lambda b, i: (b, 0, 0)),
            pl.BlockSpec((1, HID, HID), lambda b, i: (b, 0, 0)),
            pl.BlockSpec((1, NUM_LAYERS, 1, HID), lambda b, i: (b, 0, 0, 0)),
            pl.BlockSpec((1, HID, NUM_CLASSES), lambda b, i: (b, 0, 0)),
        ],
        out_specs=pl.BlockSpec((1, r, NUM_CLASSES), lambda b, i: (b, i, 0)),
        out_shape=jax.ShapeDtypeStruct((bs, num_rows, NUM_CLASSES), xs.dtype),
    )(xs[..., None], xs_lanes, w1_0[:, None], pe_proj, vc, mcat2,
      gat_lin[:, 1], bias, linf)
    return out
